# baseline (speedup 1.0000x reference)
import functools

import numpy as np
import jax
import jax.numpy as jnp
from jax import lax
from jax.experimental import pallas as pl
from jax.experimental.pallas import tpu as pltpu
from jax.sharding import Mesh, PartitionSpec as P

_ALPHA = 0.3


def _esn_fused_kernel(x_ref, h0_ref, win_ref, wr_ref, wout_ref,
                      h_seq_ref, out_ref, h_carry, pre_scratch, *, tt):
    c = pl.program_id(0)

    @pl.when(c == 0)
    def _():
        h_carry[...] = h0_ref[...]

    bb, r = h0_ref.shape
    n_in = x_ref.shape[2]

    pre_scratch[...] = jnp.dot(
        x_ref[...].reshape(tt * bb, n_in), win_ref[...],
        preferred_element_type=jnp.float32).reshape(tt, bb, r)

    wr = wr_ref[...]
    om_a = jnp.float32(1.0 - _ALPHA)
    a = jnp.float32(_ALPHA)

    def body(s, h):
        pre = pre_scratch[s] + jnp.dot(h, wr,
                                       preferred_element_type=jnp.float32)
        h_new = h * om_a + a * jnp.tanh(pre)
        h_seq_ref[s] = h_new
        return h_new

    h_final = lax.fori_loop(0, tt, body, h_carry[...], unroll=True)
    h_carry[...] = h_final

    out_ref[...] = jnp.dot(
        h_seq_ref[...].reshape(tt * bb, r).astype(jnp.bfloat16),
        wout_ref[...],
        preferred_element_type=jnp.float32).reshape(tt, bb, n_in)


def _esn_local(x_bf, h0, win_bf, wr_t, wout_bf):
    T, bb, n_in = x_bf.shape
    R = h0.shape[-1]
    tt = 8
    nc = T // tt

    h_seq, out_seq = pl.pallas_call(
        functools.partial(_esn_fused_kernel, tt=tt),
        out_shape=[
            jax.ShapeDtypeStruct((T, bb, R), jnp.float32),
            jax.ShapeDtypeStruct((T, bb, n_in), jnp.float32),
        ],
        grid=(nc,),
        in_specs=[
            pl.BlockSpec((tt, bb, n_in), lambda c: (c, 0, 0)),
            pl.BlockSpec((bb, R), lambda c: (0, 0)),
            pl.BlockSpec((n_in, R), lambda c: (0, 0)),
            pl.BlockSpec((R, R), lambda c: (0, 0)),
            pl.BlockSpec((R, n_in), lambda c: (0, 0)),
        ],
        out_specs=[
            pl.BlockSpec((tt, bb, R), lambda c: (c, 0, 0)),
            pl.BlockSpec((tt, bb, n_in), lambda c: (c, 0, 0)),
        ],
        scratch_shapes=[
            pltpu.VMEM((bb, R), jnp.float32),
            pltpu.VMEM((tt, bb, R), jnp.float32),
        ],
        compiler_params=pltpu.CompilerParams(
            dimension_semantics=("arbitrary",)),
    )(x_bf, h0, win_bf, wr_t, wout_bf)
    return out_seq, h_seq


@jax.jit
def _esn_forward(x_seq, h0, win_t, wr_t, wout_t):
    B = h0.shape[0]
    x_bf = x_seq.astype(jnp.bfloat16)
    win_bf = win_t.astype(jnp.bfloat16)
    wout_bf = wout_t.astype(jnp.bfloat16)

    devs = jax.devices()
    n_shards = len(devs) if (len(devs) > 1 and B % len(devs) == 0) else 1
    if n_shards == 1:
        return _esn_local(x_bf, h0, win_bf, wr_t, wout_bf)

    mesh = Mesh(np.array(devs[:n_shards]), ("b",))
    fn = jax.shard_map(
        _esn_local, mesh=mesh,
        in_specs=(P(None, "b", None), P("b", None),
                  P(None, None), P(None, None), P(None, None)),
        out_specs=(P(None, "b", None), P(None, "b", None)),
        check_vma=False,
    )
    return fn(x_bf, h0, win_bf, wr_t, wout_bf)


def kernel(x_seq, h0, win_t, wr_t, wout_t):
    return _esn_forward(x_seq, h0, win_t, wr_t, wout_t)


# speedup vs baseline: 4.1029x; 4.1029x over previous
import functools

import jax
import jax.numpy as jnp
from jax import lax
from jax.experimental import pallas as pl
from jax.experimental.pallas import tpu as pltpu

_ALPHA = 0.3


def _esn_fused_kernel(x_ref, h0_ref, w_cat_ref, wout_ref,
                      h_seq_ref, out_ref, hx_scratch, *, tt):
    c = pl.program_id(0)

    @pl.when(c == 0)
    def _():
        hx_scratch[:, : h0_ref.shape[1]] = h0_ref[...]

    b, r = h0_ref.shape
    n_in = x_ref.shape[2]

    w_cat = w_cat_ref[...]
    om_a = jnp.float32(1.0 - _ALPHA)
    a = jnp.float32(_ALPHA)

    def body(s, _):
        hx_scratch[:, r:] = x_ref[s].astype(jnp.float32)
        pre = jnp.dot(hx_scratch[...], w_cat,
                      preferred_element_type=jnp.float32)
        h_new = hx_scratch[:, :r] * om_a + a * jnp.tanh(pre)
        h_seq_ref[s] = h_new
        hx_scratch[:, :r] = h_new
        return 0

    lax.fori_loop(0, tt, body, 0, unroll=True)

    out_ref[...] = jnp.dot(
        h_seq_ref[...].reshape(tt * b, r).astype(jnp.bfloat16),
        wout_ref[...],
        preferred_element_type=jnp.float32).reshape(tt, b, n_in)


@jax.jit
def _esn_forward(x_seq, h0, win_t, wr_t, wout_t):
    T, B, n_in = x_seq.shape
    R = h0.shape[-1]
    tt = 16
    nc = T // tt

    x_bf = x_seq.astype(jnp.bfloat16)
    win_f32 = win_t.astype(jnp.bfloat16).astype(jnp.float32)
    w_cat = jnp.concatenate([wr_t, win_f32], axis=0)
    wout_bf = wout_t.astype(jnp.bfloat16)

    h_seq, out_seq = pl.pallas_call(
        functools.partial(_esn_fused_kernel, tt=tt),
        out_shape=[
            jax.ShapeDtypeStruct((T, B, R), jnp.float32),
            jax.ShapeDtypeStruct((T, B, n_in), jnp.float32),
        ],
        grid=(nc,),
        in_specs=[
            pl.BlockSpec((tt, B, n_in), lambda c: (c, 0, 0)),
            pl.BlockSpec((B, R), lambda c: (0, 0)),
            pl.BlockSpec((R + n_in, R), lambda c: (0, 0)),
            pl.BlockSpec((R, n_in), lambda c: (0, 0)),
        ],
        out_specs=[
            pl.BlockSpec((tt, B, R), lambda c: (c, 0, 0)),
            pl.BlockSpec((tt, B, n_in), lambda c: (c, 0, 0)),
        ],
        scratch_shapes=[
            pltpu.VMEM((B, R + n_in), jnp.float32),
        ],
        compiler_params=pltpu.CompilerParams(
            dimension_semantics=("arbitrary",)),
    )(x_bf, h0, w_cat, wout_bf)
    return out_seq, h_seq


def kernel(x_seq, h0, win_t, wr_t, wout_t):
    return _esn_forward(x_seq, h0, win_t, wr_t, wout_t)
